# Initial kernel scaffold; baseline (speedup 1.0000x reference)
import jax
import jax.numpy as jnp
from jax.experimental import pallas as pl


def kernel(x, edge_index, W1, as1, ad1, b1, g1, be1, W2, as2, ad2, b2, g2, be2, W3, as3, ad3, b3, g3, be3, W4, as4, ad4, b4):
    raise NotImplementedError("write your pallas kernel here")


# speedup vs baseline: 92.9883x; 92.9883x over previous
import functools

import numpy as np

import jax
import jax.numpy as jnp
from jax import lax
from jax.experimental import pallas as pl
from jax.experimental.pallas import tpu as pltpu
from jax.experimental.pallas import tpu_sc as plsc

N_NODES = 10000
NR = 10240
CH = 128
NW = 32
_BN_C = np.float32(1.0 / np.sqrt(1.0 + 1e-5))


def _tc_first(xp, W, As, Ad):
    NRl, Din = xp.shape
    Dout = W.shape[1]
    R = 1024
    grid = NRl // R

    def body(x_ref, w_ref, as_ref, ad_ref, th_ref, tas_ref, u_ref):
        h = jnp.dot(x_ref[...], w_ref[...], preferred_element_type=jnp.float32)
        th_ref[...] = h
        tas_ref[...] = jnp.dot(h, as_ref[...], preferred_element_type=jnp.float32)
        u_ref[...] = jnp.dot(h, ad_ref[...], preferred_element_type=jnp.float32)

    return pl.pallas_call(
        body,
        grid=(grid,),
        in_specs=[
            pl.BlockSpec((R, Din), lambda i: (i, 0)),
            pl.BlockSpec((Din, Dout), lambda i: (0, 0)),
            pl.BlockSpec((Dout, 16), lambda i: (0, 0)),
            pl.BlockSpec((Dout, 16), lambda i: (0, 0)),
        ],
        out_specs=[
            pl.BlockSpec((R, Dout), lambda i: (i, 0)),
            pl.BlockSpec((R, 16), lambda i: (i, 0)),
            pl.BlockSpec((R, 16), lambda i: (i, 0)),
        ],
        out_shape=[
            jax.ShapeDtypeStruct((NRl, Dout), jnp.float32),
            jax.ShapeDtypeStruct((NRl, 16), jnp.float32),
            jax.ShapeDtypeStruct((NRl, 16), jnp.float32),
        ],
    )(xp, W, As, Ad)


def _tc_mid(acc, den, Eexp, b, g, be, W, As, Ad):
    _, NRl, Dp = acc.shape
    Dout = W.shape[1]
    R = 1024
    grid = NRl // R

    def body(a_ref, d_ref, e_ref, b_ref, g_ref, be_ref, w_ref, as_ref, ad_ref,
             th_ref, tas_ref, u_ref):
        i = pl.program_id(0)
        z = a_ref[0] + a_ref[1]
        dd = d_ref[0] + d_ref[1]
        dex = jnp.dot(dd, e_ref[...], preferred_element_type=jnp.float32)
        prev = z / (dex + 1e-16) + b_ref[...]
        prev = g_ref[...] * prev * _BN_C + be_ref[...]
        prev = jnp.where(prev > 0, prev, jnp.exp(prev) - 1.0)
        h = jnp.dot(prev, w_ref[...], preferred_element_type=jnp.float32)
        rid = i * R + lax.broadcasted_iota(jnp.int32, (R, 1), 0)
        h = jnp.where(rid < N_NODES, h, 0.0)
        th_ref[...] = h
        tas_ref[...] = jnp.dot(h, as_ref[...], preferred_element_type=jnp.float32)
        u_ref[...] = jnp.dot(h, ad_ref[...], preferred_element_type=jnp.float32)

    return pl.pallas_call(
        body,
        grid=(grid,),
        in_specs=[
            pl.BlockSpec((2, R, Dp), lambda i: (0, i, 0)),
            pl.BlockSpec((2, R, 16), lambda i: (0, i, 0)),
            pl.BlockSpec((16, Dp), lambda i: (0, 0)),
            pl.BlockSpec((1, Dp), lambda i: (0, 0)),
            pl.BlockSpec((1, Dp), lambda i: (0, 0)),
            pl.BlockSpec((1, Dp), lambda i: (0, 0)),
            pl.BlockSpec((Dp, Dout), lambda i: (0, 0)),
            pl.BlockSpec((Dout, 16), lambda i: (0, 0)),
            pl.BlockSpec((Dout, 16), lambda i: (0, 0)),
        ],
        out_specs=[
            pl.BlockSpec((R, Dout), lambda i: (i, 0)),
            pl.BlockSpec((R, 16), lambda i: (i, 0)),
            pl.BlockSpec((R, 16), lambda i: (i, 0)),
        ],
        out_shape=[
            jax.ShapeDtypeStruct((NRl, Dout), jnp.float32),
            jax.ShapeDtypeStruct((NRl, 16), jnp.float32),
            jax.ShapeDtypeStruct((NRl, 16), jnp.float32),
        ],
    )(acc, den, Eexp, b, g, be, W, As, Ad)


def _tc_final(acc, den, b4):
    _, NRl, Dp = acc.shape
    R = 1000
    grid = N_NODES // R

    def body(a_ref, d_ref, b_ref, o_ref):
        z = a_ref[0] + a_ref[1]
        dd = d_ref[0][:, 0:1] + d_ref[1][:, 0:1]
        o = z[:, :40] / (dd + 1e-16) + b_ref[...]
        m = jnp.max(o, axis=1, keepdims=True)
        o = o - m
        o_ref[...] = o - jnp.log(jnp.sum(jnp.exp(o), axis=1, keepdims=True))

    return pl.pallas_call(
        body,
        grid=(grid,),
        in_specs=[
            pl.BlockSpec((2, R, Dp), lambda i: (0, i, 0)),
            pl.BlockSpec((2, R, 16), lambda i: (0, i, 0)),
            pl.BlockSpec((1, 40), lambda i: (0, 0)),
        ],
        out_specs=pl.BlockSpec((R, 40), lambda i: (i, 0)),
        out_shape=jax.ShapeDtypeStruct((N_NODES, 40), jnp.float32),
    )(acc, den, b4)


def _sc_edge(th, tas, u, srcp, dstp, mvec, head_of_vreg):
    NRl, Dp = th.shape
    Ep = srcp.shape[0]
    CH = 64 if Dp >= 128 else 128
    epw = Ep // NW
    nch = epw // CH
    nv = Dp // 16
    rows_per_tile = NRl // 16
    nrb = rows_per_tile // CH
    mesh = plsc.VectorSubcoreMesh(core_axis_name="c", subcore_axis_name="s")

    @functools.partial(
        pl.kernel,
        mesh=mesh,
        compiler_params=pltpu.CompilerParams(use_tc_tiling_on_sc=False),
        out_type=[
            jax.ShapeDtypeStruct((2, NRl, Dp), jnp.float32),
            jax.ShapeDtypeStruct((2, NRl, 16), jnp.float32),
        ],
        scratch_types=[
            pltpu.VMEM((2, CH, Dp), jnp.float32),
            pltpu.VMEM((2, CH, 16), jnp.float32),
            pltpu.VMEM((2, CH, 16), jnp.float32),
            pltpu.VMEM((2, CH, 16), jnp.float32),
            pltpu.VMEM((2, CH), jnp.int32),
            pltpu.VMEM((2, CH), jnp.int32),
            pltpu.VMEM((16,), jnp.float32),
            pltpu.VMEM_SHARED((NRl, Dp), jnp.float32),
            pltpu.VMEM_SHARED((NRl, 16), jnp.float32),
            pltpu.SemaphoreType.DMA,
            pltpu.SemaphoreType.DMA,
        ],
    )
    def k(th_hbm, tas_hbm, u_hbm, src_hbm, dst_hbm, mv_hbm, acc_hbm, den_hbm,
          hbuf, abuf, bbuf, ebuf, sbuf, dbuf, mbuf, acc_sh, den_sh, sem0, sem1):
        cid = lax.axis_index("c")
        sid = lax.axis_index("s")
        wid = sid * 2 + cid
        zot = jnp.zeros((16,), jnp.float32)
        sems = (sem0, sem1)

        def zrow(e, _):
            for j in range(nv):
                hbuf[0, e, pl.ds(j * 16, 16)] = zot
            ebuf[0, e] = zot
            return 0

        lax.fori_loop(0, CH, zrow, 0)
        base = sid * rows_per_tile
        for kb in range(nrb):
            pltpu.sync_copy(hbuf.at[0], acc_sh.at[pl.ds(base + kb * CH, CH)])
            pltpu.sync_copy(ebuf.at[0], den_sh.at[pl.ds(base + kb * CH, CH)])
        plsc.subcore_barrier()

        pltpu.sync_copy(mv_hbm, mbuf)
        mv = mbuf[...]

        def issue(c, b):
            off = wid * epw + c * CH
            pltpu.sync_copy(src_hbm.at[pl.ds(off, CH)], sbuf.at[b])
            pltpu.sync_copy(dst_hbm.at[pl.ds(off, CH)], dbuf.at[b])
            pltpu.async_copy(th_hbm.at[sbuf.at[b]], hbuf.at[b], sems[b])
            pltpu.async_copy(tas_hbm.at[sbuf.at[b]], abuf.at[b], sems[b])
            pltpu.async_copy(u_hbm.at[dbuf.at[b]], bbuf.at[b], sems[b])

        issue(0, 0)

        def pair(c2, _):
            for b in (0, 1):
                c = c2 * 2 + b
                nb = 1 - b

                @pl.when(c + 1 < nch)
                def _():
                    issue(c + 1, nb)

                pltpu.make_async_copy(th_hbm.at[sbuf.at[b]], hbuf.at[b], sems[b]).wait()
                pltpu.make_async_copy(tas_hbm.at[sbuf.at[b]], abuf.at[b], sems[b]).wait()
                pltpu.make_async_copy(u_hbm.at[dbuf.at[b]], bbuf.at[b], sems[b]).wait()

                @plsc.parallel_loop(0, CH, step=1, unroll=4)
                def _(e):
                    t = abuf[b, e] + bbuf[b, e]
                    t = jnp.where(t > 0, t, t * 0.2)
                    ex = jnp.exp(t - mv)
                    ebuf[b, e] = ex
                    for j in range(nv):
                        sl = pl.ds(j * 16, 16)
                        hbuf[b, e, sl] = hbuf[b, e, sl] * ex[head_of_vreg[j]]

                pltpu.sync_copy(hbuf.at[b], acc_sh.at[dbuf.at[b]], add=True)
                pltpu.sync_copy(ebuf.at[b], den_sh.at[dbuf.at[b]], add=True)
            return 0

        lax.fori_loop(0, nch // 2, pair, 0)
        plsc.subcore_barrier()
        for kb in range(nrb):
            r0 = base + kb * CH
            pltpu.sync_copy(acc_sh.at[pl.ds(r0, CH)], acc_hbm.at[cid, pl.ds(r0, CH)])
            pltpu.sync_copy(den_sh.at[pl.ds(r0, CH)], den_hbm.at[cid, pl.ds(r0, CH)])

    return k(th, tas, u, srcp, dstp, mvec)


def _expand_attn(a, Dout):
    H, C = a.shape
    ap = jnp.concatenate([a.reshape(-1), jnp.zeros((Dout - H * C,), jnp.float32)])
    head_of_row = (np.arange(Dout) // C).reshape(Dout, 1)
    valid = (np.arange(Dout) < H * C).reshape(Dout, 1)
    mask = jnp.asarray((head_of_row == np.arange(16).reshape(1, 16)) & valid)
    return jnp.where(mask, ap.reshape(Dout, 1), 0.0)


def _expand_den(Dp):
    j = np.arange(Dp)
    m = np.zeros((16, Dp), np.float32)
    m[j // 16, j] = 1.0
    return jnp.asarray(m)


def _mvec(tas, u):
    mraw = jnp.max(tas, axis=0) + jnp.max(u, axis=0)
    return jnp.where(mraw > 0, mraw, 0.2 * mraw)


def kernel(x, edge_index, W1, as1, ad1, b1, g1, be1, W2, as2, ad2, b2, g2, be2,
           W3, as3, ad3, b3, g3, be3, W4, as4, ad4, b4):
    n = x.shape[0]
    e0 = edge_index.shape[1]
    etot = e0 + n
    ep = ((etot + 2 * NW * CH - 1) // (2 * NW * CH)) * (2 * NW * CH)
    loops = jnp.arange(n, dtype=edge_index.dtype)
    padi = jnp.full((ep - etot,), n, edge_index.dtype)
    srcp = jnp.concatenate([edge_index[0], loops, padi])
    dstp = jnp.concatenate([edge_index[1], loops, padi])
    xp = jnp.pad(x, ((0, NR - n), (0, 0)))

    th, tas, u = _tc_first(xp, W1, _expand_attn(as1, 128), _expand_attn(ad1, 128))
    acc, den = _sc_edge(th, tas, u, srcp, dstp, _mvec(tas, u),
                        head_of_vreg=list(range(8)))

    th, tas, u = _tc_mid(acc, den, _expand_den(128),
                         b1.reshape(1, -1), g1.reshape(1, -1), be1.reshape(1, -1),
                         W2, _expand_attn(as2, 64), _expand_attn(ad2, 64))
    acc, den = _sc_edge(th, tas, u, srcp, dstp, _mvec(tas, u),
                        head_of_vreg=list(range(4)))

    th, tas, u = _tc_mid(acc, den, _expand_den(64),
                         b2.reshape(1, -1), g2.reshape(1, -1), be2.reshape(1, -1),
                         W3, _expand_attn(as3, 32), _expand_attn(ad3, 32))
    acc, den = _sc_edge(th, tas, u, srcp, dstp, _mvec(tas, u),
                        head_of_vreg=list(range(2)))

    W4p = jnp.pad(W4, ((0, 0), (0, 8)))
    th, tas, u = _tc_mid(acc, den, _expand_den(32),
                         b3.reshape(1, -1), g3.reshape(1, -1), be3.reshape(1, -1),
                         W4p, _expand_attn(as4, 48), _expand_attn(ad4, 48))
    acc, den = _sc_edge(th, tas, u, srcp, dstp, _mvec(tas, u),
                        head_of_vreg=[0, 0, 0])

    return _tc_final(acc, den, b4.reshape(1, -1))
